# Initial kernel scaffold; baseline (speedup 1.0000x reference)
import jax
import jax.numpy as jnp
from jax.experimental import pallas as pl


def kernel(x_h, x_l, beta, gamma):
    raise NotImplementedError("write your pallas kernel here")


# speedup vs baseline: 1.2584x; 1.2584x over previous
import math

import jax
import jax.numpy as jnp
from jax.experimental import pallas as pl
from jax.experimental.pallas import tpu as pltpu

CH = 192
C1 = 48
CL = CH - C1
REPARAM = 2.0 ** (-18)
PEDESTAL = REPARAM ** 2
BETA_MIN = 1e-6
BETA_BOUND = math.sqrt(BETA_MIN + REPARAM ** 2 + PEDESTAL)
GAMMA_BOUND = REPARAM

BLK = 2048


def _gdn_body(xh_ref, xl_ref, beta_ref, gamma_ref, yh_ref, yl_ref):
    xh = jnp.tanh(xh_ref[0])
    xl = jnp.tanh(xl_ref[0])

    g = jnp.maximum(gamma_ref[...], GAMMA_BOUND)
    g = (g * g - PEDESTAL).astype(jnp.bfloat16)
    b = jnp.maximum(beta_ref[...], BETA_BOUND)
    b2 = 2.0 * (b * b - PEDESTAL)

    x2 = jnp.concatenate([xh * xh, xl * xl], axis=0).astype(jnp.bfloat16)
    norm = jnp.dot(g, x2, preferred_element_type=jnp.float32)
    r = jax.lax.rsqrt(jnp.abs(norm + b2))

    yh_ref[0] = xh * r[:C1]
    yl_ref[0] = xl * r[C1:]


def kernel(x_h, x_l, beta, gamma):
    B, _, H, W = x_h.shape
    HW = H * W
    xh = x_h.reshape(B, C1, HW)
    xl = x_l.reshape(B, CL, HW)
    beta2 = beta.reshape(CH, 1)
    nblk = HW // BLK

    yh, yl = pl.pallas_call(
        _gdn_body,
        grid=(B, nblk),
        in_specs=[
            pl.BlockSpec((1, C1, BLK), lambda i, j: (i, 0, j)),
            pl.BlockSpec((1, CL, BLK), lambda i, j: (i, 0, j)),
            pl.BlockSpec((CH, 1), lambda i, j: (0, 0)),
            pl.BlockSpec((CH, CH), lambda i, j: (0, 0)),
        ],
        out_specs=[
            pl.BlockSpec((1, C1, BLK), lambda i, j: (i, 0, j)),
            pl.BlockSpec((1, CL, BLK), lambda i, j: (i, 0, j)),
        ],
        out_shape=[
            jax.ShapeDtypeStruct((B, C1, HW), jnp.float32),
            jax.ShapeDtypeStruct((B, CL, HW), jnp.float32),
        ],
        compiler_params=pltpu.CompilerParams(
            dimension_semantics=("parallel", "arbitrary"),
        ),
    )(xh, xl, beta2, gamma)

    return yh.reshape(B, C1, H, W), yl.reshape(B, CL, H, W)


# speedup vs baseline: 2.0394x; 1.6206x over previous
import math

import jax
import jax.numpy as jnp
from jax.experimental import pallas as pl
from jax.experimental.pallas import tpu as pltpu

CH = 192
C1 = 48
CL = CH - C1
REPARAM = 2.0 ** (-18)
PEDESTAL = REPARAM ** 2
BETA_MIN = 1e-6
BETA_BOUND = math.sqrt(BETA_MIN + REPARAM ** 2 + PEDESTAL)
GAMMA_BOUND = REPARAM

W = 256
HB = 8
KR = CH * HB


def _gdn_body(xh_ref, xl_ref, bias_ref, gbig_ref, yh_ref, yl_ref):
    xh = jnp.tanh(xh_ref[...]).reshape(C1 * HB, W)
    xl = jnp.tanh(xl_ref[...]).reshape(CL * HB, W)

    x2 = jnp.concatenate([xh * xh, xl * xl], axis=0).astype(jnp.bfloat16)
    norm = jnp.dot(gbig_ref[...], x2, preferred_element_type=jnp.float32)
    r = jax.lax.rsqrt(jnp.abs(norm + bias_ref[...]))

    yh_ref[...] = (xh * r[:C1 * HB]).reshape(C1, HB, W)
    yl_ref[...] = (xl * r[C1 * HB:]).reshape(CL, HB, W)


def kernel(x_h, x_l, beta, gamma):
    B, _, H, _ = x_h.shape

    g = jnp.maximum(gamma, GAMMA_BOUND)
    g = g * g - PEDESTAL
    g_big = jnp.kron(g, jnp.eye(HB, dtype=g.dtype)).astype(jnp.bfloat16)
    b = jnp.maximum(beta, BETA_BOUND)
    b2 = 2.0 * (b * b - PEDESTAL)
    bias_big = jnp.broadcast_to(jnp.repeat(b2, HB)[:, None], (KR, W))

    return pl.pallas_call(
        _gdn_body,
        grid=(B, H // HB),
        in_specs=[
            pl.BlockSpec((None, C1, HB, W), lambda b, h: (b, 0, h, 0)),
            pl.BlockSpec((None, CL, HB, W), lambda b, h: (b, 0, h, 0)),
            pl.BlockSpec((KR, W), lambda b, h: (0, 0)),
            pl.BlockSpec((KR, KR), lambda b, h: (0, 0)),
        ],
        out_specs=[
            pl.BlockSpec((None, C1, HB, W), lambda b, h: (b, 0, h, 0)),
            pl.BlockSpec((None, CL, HB, W), lambda b, h: (b, 0, h, 0)),
        ],
        out_shape=[
            jax.ShapeDtypeStruct(x_h.shape, jnp.float32),
            jax.ShapeDtypeStruct(x_l.shape, jnp.float32),
        ],
        compiler_params=pltpu.CompilerParams(
            dimension_semantics=("parallel", "arbitrary"),
        ),
    )(x_h, x_l, bias_big, g_big)


# speedup vs baseline: 2.6555x; 1.3021x over previous
import math

import jax
import jax.numpy as jnp
from jax.experimental import pallas as pl
from jax.experimental.pallas import tpu as pltpu

CH = 192
C1 = 48
CL = CH - C1
REPARAM = 2.0 ** (-18)
PEDESTAL = REPARAM ** 2
BETA_MIN = 1e-6
BETA_BOUND = math.sqrt(BETA_MIN + REPARAM ** 2 + PEDESTAL)
GAMMA_BOUND = REPARAM

W = 256
HB = 8
KR = CH * HB


def _gdn_body(xh_ref, xl_ref, bias_ref, gbig_ref, yh_ref, yl_ref):
    xh = jnp.tanh(xh_ref[...]).reshape(C1 * HB, W)
    xl = jnp.tanh(xl_ref[...]).reshape(CL * HB, W)

    x2 = jnp.concatenate([xh * xh, xl * xl], axis=0).astype(jnp.bfloat16)
    norm = jnp.dot(gbig_ref[...], x2, preferred_element_type=jnp.float32)
    r = jax.lax.rsqrt(jnp.abs(norm + bias_ref[...]))

    yh_ref[...] = (xh * r[:C1 * HB]).reshape(C1, HB, W)
    yl_ref[...] = (xl * r[C1 * HB:]).reshape(CL, HB, W)


def kernel(x_h, x_l, beta, gamma):
    B, _, H, _ = x_h.shape

    g = jnp.maximum(gamma, GAMMA_BOUND)
    g = g * g - PEDESTAL
    r1 = jnp.broadcast_to(g.T[:, None, :], (CH, HB, CH)).reshape(KR, CH)
    c1 = r1.T
    g2 = jnp.broadcast_to(c1[:, None, :], (CH, HB, KR)).reshape(KR, KR)
    ii = jax.lax.broadcasted_iota(jnp.int32, (KR, KR), 0)
    jj = jax.lax.broadcasted_iota(jnp.int32, (KR, KR), 1)
    g_big = jnp.where((ii & (HB - 1)) == (jj & (HB - 1)), g2, 0.0)
    g_big = g_big.astype(jnp.bfloat16)
    b = jnp.maximum(beta, BETA_BOUND)
    b2 = 2.0 * (b * b - PEDESTAL)
    bias_big = jnp.broadcast_to(jnp.repeat(b2, HB)[:, None], (KR, W))

    return pl.pallas_call(
        _gdn_body,
        grid=(B, H // HB),
        in_specs=[
            pl.BlockSpec((None, C1, HB, W), lambda b, h: (b, 0, h, 0)),
            pl.BlockSpec((None, CL, HB, W), lambda b, h: (b, 0, h, 0)),
            pl.BlockSpec((KR, W), lambda b, h: (0, 0)),
            pl.BlockSpec((KR, KR), lambda b, h: (0, 0)),
        ],
        out_specs=[
            pl.BlockSpec((None, C1, HB, W), lambda b, h: (b, 0, h, 0)),
            pl.BlockSpec((None, CL, HB, W), lambda b, h: (b, 0, h, 0)),
        ],
        out_shape=[
            jax.ShapeDtypeStruct(x_h.shape, jnp.float32),
            jax.ShapeDtypeStruct(x_l.shape, jnp.float32),
        ],
        compiler_params=pltpu.CompilerParams(
            dimension_semantics=("arbitrary", "arbitrary"),
        ),
    )(x_h, x_l, bias_big, g_big)


# speedup vs baseline: 3.1017x; 1.1680x over previous
import math

import jax
import jax.numpy as jnp
from jax.experimental import pallas as pl
from jax.experimental.pallas import tpu as pltpu

CH = 192
C1 = 48
CL = CH - C1
REPARAM = 2.0 ** (-18)
PEDESTAL = REPARAM ** 2
BETA_MIN = 1e-6
BETA_BOUND = math.sqrt(BETA_MIN + REPARAM ** 2 + PEDESTAL)
GAMMA_BOUND = REPARAM

W = 256
KRON = 8
KR = CH * KRON
HB = 16


def _gdn_body(xh_ref, xl_ref, bias_ref, gbig_ref, yh_ref, yl_ref):
    for s in range(HB // KRON):
        sl = slice(s * KRON, (s + 1) * KRON)
        xh = jnp.tanh(xh_ref[:, sl, :]).reshape(C1 * KRON, W)
        xl = jnp.tanh(xl_ref[:, sl, :]).reshape(CL * KRON, W)

        x2 = jnp.concatenate([xh * xh, xl * xl], axis=0).astype(jnp.bfloat16)
        norm = jnp.dot(gbig_ref[...], x2, preferred_element_type=jnp.float32)
        r = jax.lax.rsqrt(jnp.abs(norm + bias_ref[...]))

        yh_ref[:, sl, :] = (xh * r[:C1 * KRON]).reshape(C1, KRON, W)
        yl_ref[:, sl, :] = (xl * r[C1 * KRON:]).reshape(CL, KRON, W)


def kernel(x_h, x_l, beta, gamma):
    B, _, H, _ = x_h.shape

    g = jnp.maximum(gamma, GAMMA_BOUND)
    g = g * g - PEDESTAL
    r1 = jnp.broadcast_to(g.T[:, None, :], (CH, KRON, CH)).reshape(KR, CH)
    c1 = r1.T
    g2 = jnp.broadcast_to(c1[:, None, :], (CH, KRON, KR)).reshape(KR, KR)
    ii = jax.lax.broadcasted_iota(jnp.int32, (KR, KR), 0)
    jj = jax.lax.broadcasted_iota(jnp.int32, (KR, KR), 1)
    g_big = jnp.where((ii & (KRON - 1)) == (jj & (KRON - 1)), g2, 0.0)
    g_big = g_big.astype(jnp.bfloat16)
    b = jnp.maximum(beta, BETA_BOUND)
    b2 = 2.0 * (b * b - PEDESTAL)
    bias_big = jnp.broadcast_to(jnp.repeat(b2, KRON)[:, None], (KR, W))

    return pl.pallas_call(
        _gdn_body,
        grid=(B, H // HB),
        in_specs=[
            pl.BlockSpec((None, C1, HB, W), lambda b, h: (b, 0, h, 0)),
            pl.BlockSpec((None, CL, HB, W), lambda b, h: (b, 0, h, 0)),
            pl.BlockSpec((KR, W), lambda b, h: (0, 0)),
            pl.BlockSpec((KR, KR), lambda b, h: (0, 0)),
        ],
        out_specs=[
            pl.BlockSpec((None, C1, HB, W), lambda b, h: (b, 0, h, 0)),
            pl.BlockSpec((None, CL, HB, W), lambda b, h: (b, 0, h, 0)),
        ],
        out_shape=[
            jax.ShapeDtypeStruct(x_h.shape, jnp.float32),
            jax.ShapeDtypeStruct(x_l.shape, jnp.float32),
        ],
        compiler_params=pltpu.CompilerParams(
            dimension_semantics=("arbitrary", "arbitrary"),
            vmem_limit_bytes=100 * 1024 * 1024,
        ),
    )(x_h, x_l, bias_big, g_big)


# speedup vs baseline: 3.3877x; 1.0922x over previous
import math

import jax
import jax.numpy as jnp
from jax.experimental import pallas as pl
from jax.experimental.pallas import tpu as pltpu

CH = 192
C1 = 48
CL = CH - C1
REPARAM = 2.0 ** (-18)
PEDESTAL = REPARAM ** 2
BETA_MIN = 1e-6
BETA_BOUND = math.sqrt(BETA_MIN + REPARAM ** 2 + PEDESTAL)
GAMMA_BOUND = REPARAM

W = 256
KRON = 8
KR = CH * KRON
HB = 32


def _gdn_body(xh_ref, xl_ref, bias_ref, gbig_ref, yh_ref, yl_ref):
    for s in range(HB // KRON):
        sl = slice(s * KRON, (s + 1) * KRON)
        xh = jnp.tanh(xh_ref[:, sl, :]).reshape(C1 * KRON, W)
        xl = jnp.tanh(xl_ref[:, sl, :]).reshape(CL * KRON, W)

        x2 = jnp.concatenate([xh * xh, xl * xl], axis=0).astype(jnp.bfloat16)
        norm = jnp.dot(gbig_ref[...], x2, preferred_element_type=jnp.float32)
        r = jax.lax.rsqrt(jnp.abs(norm + bias_ref[...]))

        yh_ref[:, sl, :] = (xh * r[:C1 * KRON]).reshape(C1, KRON, W)
        yl_ref[:, sl, :] = (xl * r[C1 * KRON:]).reshape(CL, KRON, W)


def kernel(x_h, x_l, beta, gamma):
    B, _, H, _ = x_h.shape

    g = jnp.maximum(gamma, GAMMA_BOUND)
    g = g * g - PEDESTAL
    r1 = jnp.broadcast_to(g.T[:, None, :], (CH, KRON, CH)).reshape(KR, CH)
    c1 = r1.T
    g2 = jnp.broadcast_to(c1[:, None, :], (CH, KRON, KR)).reshape(KR, KR)
    ii = jax.lax.broadcasted_iota(jnp.int32, (KR, KR), 0)
    jj = jax.lax.broadcasted_iota(jnp.int32, (KR, KR), 1)
    g_big = jnp.where((ii & (KRON - 1)) == (jj & (KRON - 1)), g2, 0.0)
    g_big = g_big.astype(jnp.bfloat16)
    b = jnp.maximum(beta, BETA_BOUND)
    b2 = 2.0 * (b * b - PEDESTAL)
    bias_big = jnp.broadcast_to(jnp.repeat(b2, KRON)[:, None], (KR, W))

    return pl.pallas_call(
        _gdn_body,
        grid=(B, H // HB),
        in_specs=[
            pl.BlockSpec((None, C1, HB, W), lambda b, h: (b, 0, h, 0)),
            pl.BlockSpec((None, CL, HB, W), lambda b, h: (b, 0, h, 0)),
            pl.BlockSpec((KR, W), lambda b, h: (0, 0)),
            pl.BlockSpec((KR, KR), lambda b, h: (0, 0)),
        ],
        out_specs=[
            pl.BlockSpec((None, C1, HB, W), lambda b, h: (b, 0, h, 0)),
            pl.BlockSpec((None, CL, HB, W), lambda b, h: (b, 0, h, 0)),
        ],
        out_shape=[
            jax.ShapeDtypeStruct(x_h.shape, jnp.float32),
            jax.ShapeDtypeStruct(x_l.shape, jnp.float32),
        ],
        compiler_params=pltpu.CompilerParams(
            dimension_semantics=("arbitrary", "arbitrary"),
            vmem_limit_bytes=100 * 1024 * 1024,
        ),
    )(x_h, x_l, bias_big, g_big)


# speedup vs baseline: 3.4156x; 1.0082x over previous
import math

import jax
import jax.numpy as jnp
from jax.experimental import pallas as pl
from jax.experimental.pallas import tpu as pltpu

CH = 192
C1 = 48
CL = CH - C1
REPARAM = 2.0 ** (-18)
PEDESTAL = REPARAM ** 2
BETA_MIN = 1e-6
BETA_BOUND = math.sqrt(BETA_MIN + REPARAM ** 2 + PEDESTAL)
GAMMA_BOUND = REPARAM

W = 256
KRON = 8
KR = CH * KRON
HB = 64


def _gdn_body(xh_ref, xl_ref, bias_ref, gbig_ref, yh_ref, yl_ref):
    for s in range(HB // KRON):
        sl = slice(s * KRON, (s + 1) * KRON)
        xh = jnp.tanh(xh_ref[:, sl, :]).reshape(C1 * KRON, W)
        xl = jnp.tanh(xl_ref[:, sl, :]).reshape(CL * KRON, W)

        x2 = jnp.concatenate([xh * xh, xl * xl], axis=0).astype(jnp.bfloat16)
        norm = jnp.dot(gbig_ref[...], x2, preferred_element_type=jnp.float32)
        r = jax.lax.rsqrt(jnp.abs(norm + bias_ref[...]))

        yh_ref[:, sl, :] = (xh * r[:C1 * KRON]).reshape(C1, KRON, W)
        yl_ref[:, sl, :] = (xl * r[C1 * KRON:]).reshape(CL, KRON, W)


def kernel(x_h, x_l, beta, gamma):
    B, _, H, _ = x_h.shape

    g = jnp.maximum(gamma, GAMMA_BOUND)
    g = g * g - PEDESTAL
    r1 = jnp.broadcast_to(g.T[:, None, :], (CH, KRON, CH)).reshape(KR, CH)
    c1 = r1.T
    g2 = jnp.broadcast_to(c1[:, None, :], (CH, KRON, KR)).reshape(KR, KR)
    ii = jax.lax.broadcasted_iota(jnp.int32, (KR, KR), 0)
    jj = jax.lax.broadcasted_iota(jnp.int32, (KR, KR), 1)
    g_big = jnp.where((ii & (KRON - 1)) == (jj & (KRON - 1)), g2, 0.0)
    g_big = g_big.astype(jnp.bfloat16)
    b = jnp.maximum(beta, BETA_BOUND)
    b2 = 2.0 * (b * b - PEDESTAL)
    bias_big = jnp.broadcast_to(jnp.repeat(b2, KRON)[:, None], (KR, W))

    return pl.pallas_call(
        _gdn_body,
        grid=(B, H // HB),
        in_specs=[
            pl.BlockSpec((None, C1, HB, W), lambda b, h: (b, 0, h, 0)),
            pl.BlockSpec((None, CL, HB, W), lambda b, h: (b, 0, h, 0)),
            pl.BlockSpec((KR, W), lambda b, h: (0, 0)),
            pl.BlockSpec((KR, KR), lambda b, h: (0, 0)),
        ],
        out_specs=[
            pl.BlockSpec((None, C1, HB, W), lambda b, h: (b, 0, h, 0)),
            pl.BlockSpec((None, CL, HB, W), lambda b, h: (b, 0, h, 0)),
        ],
        out_shape=[
            jax.ShapeDtypeStruct(x_h.shape, jnp.float32),
            jax.ShapeDtypeStruct(x_l.shape, jnp.float32),
        ],
        compiler_params=pltpu.CompilerParams(
            dimension_semantics=("arbitrary", "arbitrary"),
            vmem_limit_bytes=100 * 1024 * 1024,
        ),
    )(x_h, x_l, bias_big, g_big)
